# baseline (speedup 1.0000x reference)
import functools
import jax
import jax.numpy as jnp
from jax.experimental import pallas as pl
from jax.experimental.pallas import tpu as pltpu

_RADIUS = 2.5 * 0.06
_KSIZE = 16
_MAX_NN = 16


def _gather(x, idx):
    return jax.vmap(lambda xb, ib: xb[:, ib])(x, idx)


def _conv1x1(p, x):
    return jnp.einsum('oc,bcn->bon', p['w'], x) + p['b'][None, :, None]


def _batchnorm(p, x, eps=1e-5):
    m = x.mean(axis=(0, 2), keepdims=True)
    v = x.var(axis=(0, 2), keepdims=True)
    return p['g'][None, :, None] * (x - m) / jnp.sqrt(v + eps) + p['be'][None, :, None]


def _instnorm(p, x, eps=1e-5):
    m = x.mean(axis=(2, 3), keepdims=True)
    v = x.var(axis=(2, 3), keepdims=True)
    return p['g'][None, :, None, None] * (x - m) / jnp.sqrt(v + eps) + p['be'][None, :, None, None]


def _radius_nn(pos, support, radius, max_nn):
    pos_ = jax.lax.stop_gradient(pos)
    sup_ = jax.lax.stop_gradient(support)
    pn = (pos_ ** 2).sum(1)
    sn = (sup_ ** 2).sum(1)
    d2 = sn[:, :, None] + pn[:, None, :] - 2.0 * jnp.einsum('bdm,bdn->bmn', sup_, pos_)
    neg, idx = jax.lax.top_k(-d2, max_nn)
    mask = (-neg) <= radius * radius
    return idx, mask


def _sampling(pos, ratio):
    if ratio == 1.0:
        return pos
    n = pos.shape[2]
    m = max(1, int(n * ratio))
    stride = n // m
    return pos[:, :, jnp.arange(m) * stride]


def _fkaconv(p, x, points, support, idx, mask):
    pts = _gather(points, idx)
    feats = _gather(x, idx)
    pts = pts - support[:, :, :, None]
    maxi = jnp.sqrt(jax.lax.stop_gradient((pts ** 2).sum(1)).max(axis=2))
    maxi = jnp.where(maxi == 0.0, 1.0, maxi)
    pts = pts / maxi[:, None, :, None]
    mat = jax.nn.relu(_instnorm(p['in1'], jnp.einsum('kd,bdmn->bkmn', p['fc1'], pts)))
    mp = jnp.broadcast_to(mat.max(axis=3, keepdims=True), mat.shape)
    mat = jnp.concatenate([mat, mp], axis=1)
    mat = jax.nn.relu(_instnorm(p['in2'], jnp.einsum('kc,bcmn->bkmn', p['fc2'], mat)))
    mp = jnp.broadcast_to(mat.max(axis=3, keepdims=True), mat.shape)
    mat = jnp.concatenate([mat, mp], axis=1)
    mat = jax.nn.relu(jnp.einsum('kc,bcmn->bkmn', p['fc3'], mat))
    mf = mask.astype(x.dtype)[:, None, :, :]
    agg = jnp.einsum('bcmn,bkmn->bckm', feats, mat * mf) / mat.shape[3]
    co = p['w'].shape[0]
    w = p['w'].reshape(co, feats.shape[1], mat.shape[1])
    return jnp.einsum('ock,bckm->bom', w, agg)


def _resblock(p, x, pos, ratio, radius, max_nn, idx=None, mask=None):
    support = pos if ratio == 1.0 else _sampling(pos, ratio)
    if idx is None:
        idx, mask = _radius_nn(pos, support, radius, max_nn)
    xs = x
    h = jax.nn.relu(_batchnorm(p['bn0'], _conv1x1(p['cv0'], x)))
    h = jax.nn.relu(_batchnorm(p['bn1'], _fkaconv(p['cv1'], h, pos, support, idx, mask)))
    h = _batchnorm(p['bn2'], _conv1x1(p['cv2'], h))
    if 'short' in p:
        xs = _batchnorm(p['bn_s'], _conv1x1(p['short'], xs))
    if xs.shape[2] != h.shape[2]:
        xs = _gather(xs, idx).max(axis=3)
    return jax.nn.relu(h + xs), support


def _fc_body(x_ref, w_ref, b_ref, o_ref):
    o_ref[...] = jnp.dot(x_ref[...], w_ref[...].T,
                         preferred_element_type=jnp.float32) + b_ref[...]


def _fc_pallas(xo, w, b):
    return pl.pallas_call(
        _fc_body,
        out_shape=jax.ShapeDtypeStruct((xo.shape[0], w.shape[0]), jnp.float32),
    )(xo, w, b[None, :])


def kernel(pos, x, params):
    pos = jnp.transpose(pos, (0, 2, 1))
    x = jnp.transpose(x, (0, 2, 1))
    ids0, m0 = _radius_nn(pos, pos, _RADIUS, _MAX_NN)
    x0 = jax.nn.relu(_batchnorm(params['bn0'], _fkaconv(params['cv0'], x, pos, pos, ids0, m0)))
    x0, _ = _resblock(params['b01'], x0, pos, 1.0, _RADIUS, _MAX_NN, ids0, m0)
    x1, s1 = _resblock(params['b10'], x0, pos, 0.25, _RADIUS, _MAX_NN)
    x1, _ = _resblock(params['b11'], x1, s1, 1.0, 2 * _RADIUS, _MAX_NN)
    x2, s2 = _resblock(params['b20'], x1, s1, 0.25, 2 * _RADIUS, _MAX_NN)
    x2, _ = _resblock(params['b21'], x2, s2, 1.0, 4 * _RADIUS, _MAX_NN)
    x3, s3 = _resblock(params['b30'], x2, s2, 0.25, 4 * _RADIUS, _MAX_NN)
    x3, _ = _resblock(params['b31'], x3, s3, 1.0, 8 * _RADIUS, _MAX_NN)
    x4, s4 = _resblock(params['b40'], x3, s3, 0.25, 8 * _RADIUS, _MAX_NN)
    x4, _ = _resblock(params['b41'], x4, s4, 1.0, 16 * _RADIUS, _MAX_NN)
    xo = x4.mean(axis=2)
    return _fc_pallas(xo, params['fcout']['w'], params['fcout']['b'])


# speedup vs baseline: 8.1935x; 8.1935x over previous
import functools
import jax
import jax.numpy as jnp
from jax.experimental import pallas as pl
from jax.experimental.pallas import tpu as pltpu

_RADIUS = 2.5 * 0.06
_KSIZE = 16
_MAX_NN = 16


def _gather(x, idx):
    return jax.vmap(lambda xb, ib: xb[:, ib])(x, idx)


def _conv1x1(p, x):
    return jnp.einsum('oc,bcn->bon', p['w'], x) + p['b'][None, :, None]


def _batchnorm(p, x, eps=1e-5):
    m = x.mean(axis=(0, 2), keepdims=True)
    v = x.var(axis=(0, 2), keepdims=True)
    return p['g'][None, :, None] * (x - m) / jnp.sqrt(v + eps) + p['be'][None, :, None]


def _instnorm(p, x, eps=1e-5):
    m = x.mean(axis=(2, 3), keepdims=True)
    v = x.var(axis=(2, 3), keepdims=True)
    return p['g'][None, :, None, None] * (x - m) / jnp.sqrt(v + eps) + p['be'][None, :, None, None]


def _rnn_body(sup_ref, posT_ref, pn_ref, idx_ref, val_ref, d2_ref):
    sup = sup_ref[...]
    dot = jnp.dot(sup, posT_ref[...], preferred_element_type=jnp.float32)
    sn = jnp.sum(sup * sup, axis=1, keepdims=True)
    d2_ref[...] = sn + pn_ref[...] - 2.0 * dot
    shape = d2_ref.shape
    iota = jax.lax.broadcasted_iota(jnp.int32, shape, 1)
    vals, idxs = [], []
    for _ in range(_MAX_NN):
        d2v = d2_ref[...]
        mv = jnp.min(d2v, axis=1, keepdims=True)
        am = jnp.min(jnp.where(d2v == mv, iota, jnp.int32(2**30)),
                     axis=1, keepdims=True)
        vals.append(mv)
        idxs.append(am)
        d2_ref[...] = jnp.where(iota == am, jnp.float32(1e30), d2v)
    val_ref[...] = jnp.concatenate(vals, axis=1)
    idx_ref[...] = jnp.concatenate(idxs, axis=1)


def _radius_nn_topk(pos, support):
    n = pos.shape[2]
    m = support.shape[2]
    bm = 128
    np_ = -(-n // 128) * 128
    mp = -(-m // bm) * bm
    posT = jnp.zeros((8, np_), jnp.float32).at[:3, :n].set(pos[0])
    pn = jnp.full((1, np_), 1e30, jnp.float32).at[0, :n].set((pos[0] ** 2).sum(0))
    sup = jnp.zeros((mp, 8), jnp.float32).at[:m, :3].set(support[0].T)
    grid = (mp // bm,)
    idx, vals = pl.pallas_call(
        _rnn_body,
        grid=grid,
        in_specs=[
            pl.BlockSpec((bm, 8), lambda i: (i, 0)),
            pl.BlockSpec((8, np_), lambda i: (0, 0)),
            pl.BlockSpec((1, np_), lambda i: (0, 0)),
        ],
        out_specs=[
            pl.BlockSpec((bm, _MAX_NN), lambda i: (i, 0)),
            pl.BlockSpec((bm, _MAX_NN), lambda i: (i, 0)),
        ],
        out_shape=[
            jax.ShapeDtypeStruct((mp, _MAX_NN), jnp.int32),
            jax.ShapeDtypeStruct((mp, _MAX_NN), jnp.float32),
        ],
        scratch_shapes=[pltpu.VMEM((bm, np_), jnp.float32)],
    )(sup, posT, pn)
    return idx[None, :m], vals[None, :m]


def _radius_nn(pos, support, radius, max_nn):
    del max_nn
    idx, vals = _radius_nn_topk(pos, support)
    mask = vals <= radius * radius
    return idx, mask


def _sampling(pos, ratio):
    if ratio == 1.0:
        return pos
    n = pos.shape[2]
    m = max(1, int(n * ratio))
    stride = n // m
    return pos[:, :, jnp.arange(m) * stride]


def _fkaconv(p, x, points, support, idx, mask):
    pts = _gather(points, idx)
    feats = _gather(x, idx)
    pts = pts - support[:, :, :, None]
    maxi = jnp.sqrt(jax.lax.stop_gradient((pts ** 2).sum(1)).max(axis=2))
    maxi = jnp.where(maxi == 0.0, 1.0, maxi)
    pts = pts / maxi[:, None, :, None]
    mat = jax.nn.relu(_instnorm(p['in1'], jnp.einsum('kd,bdmn->bkmn', p['fc1'], pts)))
    mp = jnp.broadcast_to(mat.max(axis=3, keepdims=True), mat.shape)
    mat = jnp.concatenate([mat, mp], axis=1)
    mat = jax.nn.relu(_instnorm(p['in2'], jnp.einsum('kc,bcmn->bkmn', p['fc2'], mat)))
    mp = jnp.broadcast_to(mat.max(axis=3, keepdims=True), mat.shape)
    mat = jnp.concatenate([mat, mp], axis=1)
    mat = jax.nn.relu(jnp.einsum('kc,bcmn->bkmn', p['fc3'], mat))
    mf = mask.astype(x.dtype)[:, None, :, :]
    agg = jnp.einsum('bcmn,bkmn->bckm', feats, mat * mf) / mat.shape[3]
    co = p['w'].shape[0]
    w = p['w'].reshape(co, feats.shape[1], mat.shape[1])
    return jnp.einsum('ock,bckm->bom', w, agg)


def _resblock(p, x, pos, ratio, radius, max_nn, idx=None, mask=None):
    support = pos if ratio == 1.0 else _sampling(pos, ratio)
    if idx is None:
        idx, mask = _radius_nn(pos, support, radius, max_nn)
    xs = x
    h = jax.nn.relu(_batchnorm(p['bn0'], _conv1x1(p['cv0'], x)))
    h = jax.nn.relu(_batchnorm(p['bn1'], _fkaconv(p['cv1'], h, pos, support, idx, mask)))
    h = _batchnorm(p['bn2'], _conv1x1(p['cv2'], h))
    if 'short' in p:
        xs = _batchnorm(p['bn_s'], _conv1x1(p['short'], xs))
    if xs.shape[2] != h.shape[2]:
        xs = _gather(xs, idx).max(axis=3)
    return jax.nn.relu(h + xs), support


def _fc_body(x_ref, w_ref, b_ref, o_ref):
    o_ref[...] = jnp.dot(x_ref[...], w_ref[...].T,
                         preferred_element_type=jnp.float32) + b_ref[...]


def _fc_pallas(xo, w, b):
    return pl.pallas_call(
        _fc_body,
        out_shape=jax.ShapeDtypeStruct((xo.shape[0], w.shape[0]), jnp.float32),
    )(xo, w, b[None, :])


def kernel(pos, x, params):
    pos = jnp.transpose(pos, (0, 2, 1))
    x = jnp.transpose(x, (0, 2, 1))
    ids0, m0 = _radius_nn(pos, pos, _RADIUS, _MAX_NN)
    x0 = jax.nn.relu(_batchnorm(params['bn0'], _fkaconv(params['cv0'], x, pos, pos, ids0, m0)))
    x0, _ = _resblock(params['b01'], x0, pos, 1.0, _RADIUS, _MAX_NN, ids0, m0)
    x1, s1 = _resblock(params['b10'], x0, pos, 0.25, _RADIUS, _MAX_NN)
    x1, _ = _resblock(params['b11'], x1, s1, 1.0, 2 * _RADIUS, _MAX_NN)
    x2, s2 = _resblock(params['b20'], x1, s1, 0.25, 2 * _RADIUS, _MAX_NN)
    x2, _ = _resblock(params['b21'], x2, s2, 1.0, 4 * _RADIUS, _MAX_NN)
    x3, s3 = _resblock(params['b30'], x2, s2, 0.25, 4 * _RADIUS, _MAX_NN)
    x3, _ = _resblock(params['b31'], x3, s3, 1.0, 8 * _RADIUS, _MAX_NN)
    x4, s4 = _resblock(params['b40'], x3, s3, 0.25, 8 * _RADIUS, _MAX_NN)
    x4, _ = _resblock(params['b41'], x4, s4, 1.0, 16 * _RADIUS, _MAX_NN)
    xo = x4.mean(axis=2)
    return _fc_pallas(xo, params['fcout']['w'], params['fcout']['b'])


# speedup vs baseline: 15.1286x; 1.8464x over previous
import functools
import jax
import jax.numpy as jnp
from jax import lax
from jax.experimental import pallas as pl
from jax.experimental.pallas import tpu as pltpu
from jax.experimental.pallas import tpu_sc as plsc

_RADIUS = 2.5 * 0.06
_MAX_NN = 16


_NW = 32


def _gchunk(d):
    return 128 if d <= 512 else 64


def _sc_gather_rows(table, idxf):
    n, d = table.shape
    chunk = _gchunk(d)
    b = idxf.shape[0]
    bpw = b // _NW
    nch = bpw // chunk
    mesh = plsc.VectorSubcoreMesh(core_axis_name="c", subcore_axis_name="s")

    @functools.partial(
        pl.kernel, mesh=mesh,
        compiler_params=pltpu.CompilerParams(use_tc_tiling_on_sc=False),
        out_type=jax.ShapeDtypeStruct((b, d), jnp.float32),
        scratch_types=[
            pltpu.VMEM((chunk,), jnp.int32),
            pltpu.VMEM((chunk, d), jnp.float32),
            pltpu.SemaphoreType.DMA,
        ],
    )
    def gk(table_hbm, idx_hbm, out_hbm, idx_v, rows_v, sem):
        wid = lax.axis_index("s") * 2 + lax.axis_index("c")
        base = wid * bpw

        def body(j, carry):
            off = base + j * chunk
            pltpu.sync_copy(idx_hbm.at[pl.ds(off, chunk)], idx_v)
            pltpu.async_copy(table_hbm.at[idx_v], rows_v, sem).wait()
            pltpu.sync_copy(rows_v, out_hbm.at[pl.ds(off, chunk)])
            return carry

        lax.fori_loop(0, nch, body, 0)

    return gk(table, idxf)


def _gather(x, idx):
    n, c = x.shape[1], x.shape[2]
    m, k = idx.shape[1], idx.shape[2]
    dp = -(-c // 16) * 16
    ch = _gchunk(dp)
    bp = -(-(m * k) // (_NW * ch)) * (_NW * ch)
    if dp == c:
        table = x[0]
    else:
        table = jnp.zeros((n, dp), jnp.float32).at[:, :c].set(x[0])
    idxf = jnp.zeros((bp,), jnp.int32).at[:m * k].set(idx[0].reshape(-1))
    out = _sc_gather_rows(table, idxf)
    return out[:m * k, :c].reshape(1, m, k, c)


def _rnn_body(sup_ref, posT_ref, pn_ref, idx_ref, val_ref, d2_ref):
    sup = sup_ref[...]
    dot = jnp.dot(sup, posT_ref[...], preferred_element_type=jnp.float32)
    sn = jnp.sum(sup * sup, axis=1, keepdims=True)
    d2_ref[...] = sn + pn_ref[...] - 2.0 * dot
    shape = d2_ref.shape
    iota = lax.broadcasted_iota(jnp.int32, shape, 1)
    vals, idxs = [], []
    for _ in range(_MAX_NN):
        d2v = d2_ref[...]
        mv = jnp.min(d2v, axis=1, keepdims=True)
        am = jnp.min(jnp.where(d2v == mv, iota, jnp.int32(2**30)),
                     axis=1, keepdims=True)
        vals.append(mv)
        idxs.append(am)
        d2_ref[...] = jnp.where(iota == am, jnp.float32(1e30), d2v)
    val_ref[...] = jnp.concatenate(vals, axis=1)
    idx_ref[...] = jnp.concatenate(idxs, axis=1)


def _radius_nn_topk(pos, support):
    n = pos.shape[1]
    m = support.shape[1]
    bm = 128
    np_ = -(-n // 128) * 128
    mp = -(-m // bm) * bm
    posT = jnp.zeros((8, np_), jnp.float32).at[:3, :n].set(pos[0].T)
    pn = jnp.full((1, np_), 1e30, jnp.float32).at[0, :n].set((pos[0] ** 2).sum(1))
    sup = jnp.zeros((mp, 8), jnp.float32).at[:m, :3].set(support[0])
    grid = (mp // bm,)
    idx, vals = pl.pallas_call(
        _rnn_body,
        grid=grid,
        in_specs=[
            pl.BlockSpec((bm, 8), lambda i: (i, 0)),
            pl.BlockSpec((8, np_), lambda i: (0, 0)),
            pl.BlockSpec((1, np_), lambda i: (0, 0)),
        ],
        out_specs=[
            pl.BlockSpec((bm, _MAX_NN), lambda i: (i, 0)),
            pl.BlockSpec((bm, _MAX_NN), lambda i: (i, 0)),
        ],
        out_shape=[
            jax.ShapeDtypeStruct((mp, _MAX_NN), jnp.int32),
            jax.ShapeDtypeStruct((mp, _MAX_NN), jnp.float32),
        ],
        scratch_shapes=[pltpu.VMEM((bm, np_), jnp.float32)],
    )(sup, posT, pn)
    return idx[None, :m], vals[None, :m]


def _radius_nn(pos, support, radius):
    idx, vals = _radius_nn_topk(pos, support)
    mask = vals <= radius * radius
    return idx, mask, vals


def _conv1x1(p, x):
    return jnp.dot(x, p['w'].T, preferred_element_type=jnp.float32) + p['b']


def _batchnorm(p, x, eps=1e-5):
    m = x.mean(axis=(0, 1), keepdims=True)
    v = x.var(axis=(0, 1), keepdims=True)
    return p['g'] * (x - m) / jnp.sqrt(v + eps) + p['be']


def _instnorm(p, x, eps=1e-5):
    m = x.mean(axis=(1, 2), keepdims=True)
    v = x.var(axis=(1, 2), keepdims=True)
    return p['g'] * (x - m) / jnp.sqrt(v + eps) + p['be']


def _sampling(pos, ratio):
    if ratio == 1.0:
        return pos
    n = pos.shape[1]
    m = max(1, int(n * ratio))
    stride = n // m
    return pos[:, jnp.arange(m) * stride, :]


def _fkaconv(p, x, points, support, idx, mask, vals):
    k = idx.shape[2]
    pts = _gather(points, idx)
    pts = pts - support[:, :, None, :]
    maxi = jnp.sqrt(jnp.maximum(jnp.max(vals, axis=2), 0.0))
    maxi = jnp.where(maxi == 0.0, 1.0, maxi)
    pts = pts / maxi[:, :, None, None]
    mat = jax.nn.relu(_instnorm(p['in1'], jnp.dot(pts, p['fc1'].T)))
    mp = jnp.broadcast_to(mat.max(axis=2, keepdims=True), mat.shape)
    mat = jnp.concatenate([mat, mp], axis=-1)
    mat = jax.nn.relu(_instnorm(p['in2'], jnp.dot(mat, p['fc2'].T)))
    mp = jnp.broadcast_to(mat.max(axis=2, keepdims=True), mat.shape)
    mat = jnp.concatenate([mat, mp], axis=-1)
    mat = jax.nn.relu(jnp.dot(mat, p['fc3'].T))
    feats = _gather(x, idx)
    matm = mat * mask.astype(x.dtype)[..., None] / k
    agg = jnp.einsum('bmnc,bmnk->bmck', feats, matm)
    co = p['w'].shape[0]
    w = p['w'].reshape(co, feats.shape[3], mat.shape[3])
    return jnp.einsum('ock,bmck->bmo', w, agg)


def _resblock(p, x, pos, ratio, radius, idx=None, mask=None, vals=None):
    support = pos if ratio == 1.0 else _sampling(pos, ratio)
    if idx is None:
        idx, mask, vals = _radius_nn(pos, support, radius)
    xs = x
    h = jax.nn.relu(_batchnorm(p['bn0'], _conv1x1(p['cv0'], x)))
    h = jax.nn.relu(_batchnorm(p['bn1'], _fkaconv(p['cv1'], h, pos, support, idx, mask, vals)))
    h = _batchnorm(p['bn2'], _conv1x1(p['cv2'], h))
    if 'short' in p:
        xs = _batchnorm(p['bn_s'], _conv1x1(p['short'], xs))
    if xs.shape[1] != h.shape[1]:
        xs = _gather(xs, idx).max(axis=2)
    return jax.nn.relu(h + xs), support


def _fc_body(x_ref, w_ref, b_ref, o_ref):
    o_ref[...] = jnp.dot(x_ref[...], w_ref[...].T,
                         preferred_element_type=jnp.float32) + b_ref[...]


def _fc_pallas(xo, w, b):
    return pl.pallas_call(
        _fc_body,
        out_shape=jax.ShapeDtypeStruct((xo.shape[0], w.shape[0]), jnp.float32),
    )(xo, w, b[None, :])


def kernel(pos, x, params):
    ids0, m0, v0 = _radius_nn(pos, pos, _RADIUS)
    x0 = jax.nn.relu(_batchnorm(params['bn0'],
                                _fkaconv(params['cv0'], x, pos, pos, ids0, m0, v0)))
    x0, _ = _resblock(params['b01'], x0, pos, 1.0, _RADIUS, ids0, m0, v0)
    x1, s1 = _resblock(params['b10'], x0, pos, 0.25, _RADIUS)
    x1, _ = _resblock(params['b11'], x1, s1, 1.0, 2 * _RADIUS)
    x2, s2 = _resblock(params['b20'], x1, s1, 0.25, 2 * _RADIUS)
    x2, _ = _resblock(params['b21'], x2, s2, 1.0, 4 * _RADIUS)
    x3, s3 = _resblock(params['b30'], x2, s2, 0.25, 4 * _RADIUS)
    x3, _ = _resblock(params['b31'], x3, s3, 1.0, 8 * _RADIUS)
    x4, s4 = _resblock(params['b40'], x3, s3, 0.25, 8 * _RADIUS)
    x4, _ = _resblock(params['b41'], x4, s4, 1.0, 16 * _RADIUS)
    xo = x4.mean(axis=1)
    return _fc_pallas(xo, params['fcout']['w'], params['fcout']['b'])
